# Initial kernel scaffold; baseline (speedup 1.0000x reference)
import jax
import jax.numpy as jnp
from jax.experimental import pallas as pl


def kernel(x, edge_index, edge_type, basis_V, basis_coeff, bias, self_loop_weight):
    raise NotImplementedError("write your pallas kernel here")


# speedup vs baseline: 4.1733x; 4.1733x over previous
import jax
import jax.numpy as jnp
from jax import lax
from jax.experimental import pallas as pl
from jax.experimental.pallas import tpu as pltpu
from jax.experimental.pallas import tpu_sc as plsc

N = 10000
E = 320000
D = 128
R = 5
NB = 2

NS = 16
CHUNK = 128
NCHUNKS = E // CHUNK
KMAX = -(-NCHUNKS // NS)
DEG_PAD = 51200


def _sc_body(x_hbm, ei_hbm, et_hbm, cf_hbm, u_hbm,
             src_buf, dst_buf, etype_buf, idx_buf, ones_buf, dg_buf,
             rows, rows_s, s_buf, coeff_buf, zero1d, u_sh, deg_sh,
             sem, sem2):
    b = lax.axis_index("c")
    t = lax.axis_index("s")

    zf = jnp.zeros((16,), jnp.float32)
    of = jnp.ones((16,), jnp.float32)
    for i in range(CHUNK // 16):
        ones_buf[pl.ds(i * 16, 16)] = of
    for i in range(3200 // 16):
        zero1d[pl.ds(i * 16, 16)] = zf

    def _zero_rows(i, carry):
        for seg in range(8):
            rows_s[i, pl.ds(seg * 16, 16)] = zf
        return carry
    lax.fori_loop(0, CHUNK, _zero_rows, 0)

    def _zero_u(k, carry):
        c = k * NS + t

        @pl.when(c < N // 80)
        def _():
            pltpu.sync_copy(rows_s.at[pl.ds(0, 80)],
                            u_sh.at[pl.ds(c * 80, 80)])
        return carry
    lax.fori_loop(0, -(-(N // 80) // NS), _zero_u, 0)
    pltpu.sync_copy(zero1d, deg_sh.at[pl.ds(t * 3200, 3200)])
    pltpu.sync_copy(cf_hbm, coeff_buf)
    plsc.subcore_barrier()

    def _p1(k, carry):
        c = k * NS + t

        @pl.when(c < NCHUNKS)
        def _():
            off = c * CHUNK
            pltpu.sync_copy(ei_hbm.at[1, pl.ds(off, CHUNK)], dst_buf)
            pltpu.sync_copy(et_hbm.at[pl.ds(off, CHUNK)], etype_buf)
            for seg in range(8):
                d = dst_buf[pl.ds(seg * 16, 16)]
                r = etype_buf[pl.ds(seg * 16, 16)]
                idx_buf[pl.ds(seg * 16, 16)] = r * N + d
            pltpu.sync_copy(ones_buf, deg_sh.at[idx_buf], add=True)
        return carry
    lax.fori_loop(0, KMAX, _p1, 0)
    plsc.subcore_barrier()

    bvec = jnp.full((16,), b, jnp.int32)

    def _p2(k, carry):
        c = k * NS + t

        @pl.when(c < NCHUNKS)
        def _():
            off = c * CHUNK
            pltpu.sync_copy(ei_hbm.at[0, pl.ds(off, CHUNK)], src_buf)
            pltpu.sync_copy(ei_hbm.at[1, pl.ds(off, CHUNK)], dst_buf)
            pltpu.sync_copy(et_hbm.at[pl.ds(off, CHUNK)], etype_buf)
            for seg in range(8):
                d = dst_buf[pl.ds(seg * 16, 16)]
                r = etype_buf[pl.ds(seg * 16, 16)]
                idx_buf[pl.ds(seg * 16, 16)] = r * N + d
            cp_rows = pltpu.async_copy(x_hbm.at[src_buf], rows, sem)
            cp_deg = pltpu.async_copy(deg_sh.at[idx_buf], dg_buf, sem2)
            cp_rows.wait()
            cp_deg.wait()
            for seg in range(8):
                r = etype_buf[pl.ds(seg * 16, 16)]
                dg = dg_buf[pl.ds(seg * 16, 16)]
                cf = plsc.load_gather(coeff_buf, [r * NB + bvec])
                s_buf[pl.ds(seg * 16, 16)] = cf / jnp.maximum(dg, 1.0)

            def _scale(e, cc):
                sv = plsc.load_gather(s_buf, [jnp.full((16,), e, jnp.int32)])
                for seg in range(8):
                    rows_s[e, pl.ds(seg * 16, 16)] = (
                        rows[e, pl.ds(seg * 16, 16)] * sv)
                return cc
            lax.fori_loop(0, CHUNK, _scale, 0)
            pltpu.sync_copy(rows_s, u_sh.at[dst_buf], add=True)
        return carry
    lax.fori_loop(0, KMAX, _p2, 0)
    plsc.subcore_barrier()

    def _ep(k, carry):
        c = k * NS + t

        @pl.when(c < N // 80)
        def _():
            pltpu.sync_copy(u_sh.at[pl.ds(c * 80, 80)],
                            u_hbm.at[b, pl.ds(c * 80, 80)])
        return carry
    lax.fori_loop(0, -(-(N // 80) // NS), _ep, 0)


_sc_call = pl.kernel(
    _sc_body,
    out_type=jax.ShapeDtypeStruct((NB, N, D), jnp.float32),
    mesh=plsc.VectorSubcoreMesh(core_axis_name="c", subcore_axis_name="s"),
    compiler_params=pltpu.CompilerParams(needs_layout_passes=False),
    scratch_types=[
        pltpu.VMEM((CHUNK,), jnp.int32),
        pltpu.VMEM((CHUNK,), jnp.int32),
        pltpu.VMEM((CHUNK,), jnp.int32),
        pltpu.VMEM((CHUNK,), jnp.int32),
        pltpu.VMEM((CHUNK,), jnp.float32),
        pltpu.VMEM((CHUNK,), jnp.float32),
        pltpu.VMEM((CHUNK, D), jnp.float32),
        pltpu.VMEM((CHUNK, D), jnp.float32),
        pltpu.VMEM((CHUNK,), jnp.float32),
        pltpu.VMEM((16,), jnp.float32),
        pltpu.VMEM((3200,), jnp.float32),
        pltpu.VMEM_SHARED((N, D), jnp.float32),
        pltpu.VMEM_SHARED((DEG_PAD,), jnp.float32),
        pltpu.SemaphoreType.DMA,
        pltpu.SemaphoreType.DMA,
    ],
)


BLK = 1000


def _tc_body(u_ref, x_ref, v_ref, wsl_ref, bias_ref, o_ref):
    acc = jnp.dot(u_ref[0], v_ref[0], preferred_element_type=jnp.float32)
    acc = acc + jnp.dot(u_ref[1], v_ref[1], preferred_element_type=jnp.float32)
    acc = acc + jnp.dot(x_ref[...], wsl_ref[...],
                        preferred_element_type=jnp.float32)
    o_ref[...] = acc + bias_ref[...]


_tc_call = pl.pallas_call(
    _tc_body,
    grid=(N // BLK,),
    in_specs=[
        pl.BlockSpec((NB, BLK, D), lambda i: (0, i, 0)),
        pl.BlockSpec((BLK, D), lambda i: (i, 0)),
        pl.BlockSpec((NB, D, D), lambda i: (0, 0, 0)),
        pl.BlockSpec((D, D), lambda i: (0, 0)),
        pl.BlockSpec((1, D), lambda i: (0, 0)),
    ],
    out_specs=pl.BlockSpec((BLK, D), lambda i: (i, 0)),
    out_shape=jax.ShapeDtypeStruct((N, D), jnp.float32),
)


@jax.jit
def kernel(x, edge_index, edge_type, basis_V, basis_coeff, bias,
           self_loop_weight):
    coeff_flat = jnp.pad(basis_coeff.reshape(-1), (0, 16 - R * NB))
    u = _sc_call(x, edge_index, edge_type, coeff_flat)
    return _tc_call(u, x, basis_V, self_loop_weight, bias.reshape(1, D))


# speedup vs baseline: 10.7362x; 2.5726x over previous
import jax
import jax.numpy as jnp
from jax import lax
from jax.experimental import pallas as pl
from jax.experimental.pallas import tpu as pltpu
from jax.experimental.pallas import tpu_sc as plsc

N = 10000
E = 320000
D = 128
R = 5
NB = 2

NS = 16
EPT = E // NS
SUP = 400
CHK = 80
CPS = SUP // CHK
NSUP = EPT // SUP
DEG_PAD = 51200


def _sc_body(x_hbm, e3_hbm, cf_hbm, u_hbm,
             sup0, sup1, rows0, src0, dst0, idx0, dg0, sb0,
             ones_buf, coeff_buf, zero1d, u_sh, deg_sh,
             sem_ei0, sem_ei1, sem_dg, sem_g):
    b = lax.axis_index("c")
    t = lax.axis_index("s")
    base = t * EPT

    sup = [sup0, sup1]
    sem_ei = [sem_ei0, sem_ei1]

    zf = jnp.zeros((16,), jnp.float32)
    of = jnp.ones((16,), jnp.float32)
    for i in range(CHK // 16):
        ones_buf[pl.ds(i * 16, 16)] = of
    for i in range(3200 // 16):
        zero1d[pl.ds(i * 16, 16)] = zf

    def _zero_rows(i, carry):
        for seg in range(8):
            rows0[i, pl.ds(seg * 16, 16)] = zf
        return carry
    lax.fori_loop(0, CHK, _zero_rows, 0)

    def _zero_u(k, carry):
        c = k * NS + t

        @pl.when(c < N // CHK)
        def _():
            pltpu.sync_copy(rows0, u_sh.at[pl.ds(c * CHK, CHK)])
        return carry
    lax.fori_loop(0, -(-(N // CHK) // NS), _zero_u, 0)
    pltpu.sync_copy(zero1d, deg_sh.at[pl.ds(t * 3200, 3200)])
    pltpu.sync_copy(cf_hbm, coeff_buf)
    plsc.subcore_barrier()

    def _load_sup(s, sp):
        off = base + s * SUP
        for rix in range(3):
            pltpu.async_copy(
                e3_hbm.at[pl.ds(rix * E + off, SUP)],
                sup[sp].at[pl.ds(rix * SUP, SUP)], sem_ei[sp])

    def _wait_sup(sp):
        for rix in range(3):
            pltpu.make_async_copy(
                e3_hbm.at[pl.ds(0, SUP)],
                sup[sp].at[pl.ds(rix * SUP, SUP)], sem_ei[sp]).wait()

    _load_sup(0, 0)

    def _p1(u, carry):
        for sph in range(2):
            s = 2 * u + sph
            _wait_sup(sph)

            @pl.when(s + 1 < NSUP)
            def _():
                _load_sup(s + 1, 1 - sph)
            for j in range(CPS):
                for seg in range(CHK // 16):
                    cbase = j * CHK + seg * 16
                    d = sup[sph][pl.ds(SUP + cbase, 16)]
                    r = sup[sph][pl.ds(2 * SUP + cbase, 16)]
                    idx0[pl.ds(seg * 16, 16)] = r * N + d
                pltpu.sync_copy(ones_buf, deg_sh.at[idx0], add=True)
        return carry
    lax.fori_loop(0, NSUP // 2, _p1, 0)
    plsc.subcore_barrier()

    bvec = jnp.full((16,), b, jnp.int32)
    _load_sup(0, 0)

    def _p2(u, carry):
        for sph in range(2):
            s = 2 * u + sph
            _wait_sup(sph)

            @pl.when(s + 1 < NSUP)
            def _():
                _load_sup(s + 1, 1 - sph)
            for j in range(CPS):
                for seg in range(CHK // 16):
                    cbase = j * CHK + seg * 16
                    sv = sup[sph][pl.ds(cbase, 16)]
                    d = sup[sph][pl.ds(SUP + cbase, 16)]
                    r = sup[sph][pl.ds(2 * SUP + cbase, 16)]
                    src0[pl.ds(seg * 16, 16)] = sv
                    dst0[pl.ds(seg * 16, 16)] = d
                    idx0[pl.ds(seg * 16, 16)] = r * N + d
                    cf = plsc.load_gather(coeff_buf, [r * NB + bvec])
                    sb0[pl.ds(seg * 16, 16)] = cf
                cp_dg = pltpu.async_copy(deg_sh.at[idx0], dg0, sem_dg)
                cp_g = pltpu.async_copy(x_hbm.at[src0], rows0, sem_g)
                cp_dg.wait()
                cp_g.wait()
                for seg in range(CHK // 16):
                    sl = pl.ds(seg * 16, 16)
                    sb0[sl] = sb0[sl] / jnp.maximum(dg0[sl], 1.0)

                def _scale(e):
                    svv = plsc.load_gather(
                        sb0, [jnp.full((16,), e, jnp.int32)])
                    for seg in range(8):
                        rows0[e, pl.ds(seg * 16, 16)] = (
                            rows0[e, pl.ds(seg * 16, 16)] * svv)
                plsc.parallel_loop(0, CHK, unroll=4)(_scale)
                pltpu.sync_copy(rows0, u_sh.at[dst0], add=True)
        return carry
    lax.fori_loop(0, NSUP // 2, _p2, 0)
    plsc.subcore_barrier()

    def _ep(k, carry):
        c = k * NS + t

        @pl.when(c < N // CHK)
        def _():
            pltpu.sync_copy(u_sh.at[pl.ds(c * CHK, CHK)],
                            u_hbm.at[b, pl.ds(c * CHK, CHK)])
        return carry
    lax.fori_loop(0, -(-(N // CHK) // NS), _ep, 0)


_sc_call = pl.kernel(
    _sc_body,
    out_type=jax.ShapeDtypeStruct((NB, N, D), jnp.float32),
    mesh=plsc.VectorSubcoreMesh(core_axis_name="c", subcore_axis_name="s"),
    compiler_params=pltpu.CompilerParams(needs_layout_passes=False),
    scratch_types=[
        pltpu.VMEM((3 * SUP,), jnp.int32),
        pltpu.VMEM((3 * SUP,), jnp.int32),
        pltpu.VMEM((CHK, D), jnp.float32),
        pltpu.VMEM((CHK,), jnp.int32),
        pltpu.VMEM((CHK,), jnp.int32),
        pltpu.VMEM((CHK,), jnp.int32),
        pltpu.VMEM((CHK,), jnp.float32),
        pltpu.VMEM((CHK,), jnp.float32),
        pltpu.VMEM((CHK,), jnp.float32),
        pltpu.VMEM((16,), jnp.float32),
        pltpu.VMEM((3200,), jnp.float32),
        pltpu.VMEM_SHARED((N, D), jnp.float32),
        pltpu.VMEM_SHARED((DEG_PAD,), jnp.float32),
        pltpu.SemaphoreType.DMA,
        pltpu.SemaphoreType.DMA,
        pltpu.SemaphoreType.DMA,
        pltpu.SemaphoreType.DMA,
    ],
)


BLK = 1000


def _tc_body(u_ref, x_ref, v_ref, wsl_ref, bias_ref, o_ref):
    acc = jnp.dot(u_ref[0], v_ref[0], preferred_element_type=jnp.float32)
    acc = acc + jnp.dot(u_ref[1], v_ref[1], preferred_element_type=jnp.float32)
    acc = acc + jnp.dot(x_ref[...], wsl_ref[...],
                        preferred_element_type=jnp.float32)
    o_ref[...] = acc + bias_ref[...]


_tc_call = pl.pallas_call(
    _tc_body,
    grid=(N // BLK,),
    in_specs=[
        pl.BlockSpec((NB, BLK, D), lambda i: (0, i, 0)),
        pl.BlockSpec((BLK, D), lambda i: (i, 0)),
        pl.BlockSpec((NB, D, D), lambda i: (0, 0, 0)),
        pl.BlockSpec((D, D), lambda i: (0, 0)),
        pl.BlockSpec((1, D), lambda i: (0, 0)),
    ],
    out_specs=pl.BlockSpec((BLK, D), lambda i: (i, 0)),
    out_shape=jax.ShapeDtypeStruct((N, D), jnp.float32),
)


@jax.jit
def kernel(x, edge_index, edge_type, basis_V, basis_coeff, bias,
           self_loop_weight):
    edges3 = jnp.concatenate([edge_index, edge_type[None, :]],
                             axis=0).reshape(-1)
    coeff_flat = jnp.pad(basis_coeff.reshape(-1), (0, 16 - R * NB))
    u = _sc_call(x, edges3, coeff_flat)
    return _tc_call(u, x, basis_V, self_loop_weight, bias.reshape(1, D))


# speedup vs baseline: 17.3664x; 1.6176x over previous
import jax
import jax.numpy as jnp
from jax import lax
from jax.experimental import pallas as pl
from jax.experimental.pallas import tpu as pltpu
from jax.experimental.pallas import tpu_sc as plsc

N = 10000
E = 320000
D = 128
R = 5
NB = 2

NS = 16
EPT = E // NS
SUP = 2000
CHK = 80
CPS = SUP // CHK
NSUP = EPT // SUP
DEG_PAD = 51200


def _sc_body(x_hbm, e3_hbm, cf_hbm, u_hbm,
             sup0, rows0, rows1, src0, src1, dst0, dst1, idx0, idx1,
             dg0, dg1, sb0, sb1, ones_buf, coeff_buf, zero1d, u_sh, deg_sh,
             sem_ei0, sem_dg0, sem_dg1, sem_g0, sem_g1,
             sem_sc0, sem_sc1, sem_d10, sem_d11):
    b = lax.axis_index("c")
    t = lax.axis_index("s")
    base = t * EPT

    rows = [rows0, rows1]
    src = [src0, src1]
    dst = [dst0, dst1]
    idx = [idx0, idx1]
    dg = [dg0, dg1]
    sb = [sb0, sb1]
    sem_dg = [sem_dg0, sem_dg1]
    sem_g = [sem_g0, sem_g1]
    sem_sc = [sem_sc0, sem_sc1]
    sem_d1 = [sem_d10, sem_d11]

    zf = jnp.zeros((16,), jnp.float32)
    of = jnp.ones((16,), jnp.float32)
    for i in range(CHK // 16):
        ones_buf[pl.ds(i * 16, 16)] = of
    for i in range(3200 // 16):
        zero1d[pl.ds(i * 16, 16)] = zf

    def _zero_rows(i, carry):
        for seg in range(8):
            rows0[i, pl.ds(seg * 16, 16)] = zf
        return carry
    lax.fori_loop(0, CHK, _zero_rows, 0)

    def _zero_u(k, carry):
        c = k * NS + t

        @pl.when(c < N // CHK)
        def _():
            pltpu.sync_copy(rows0, u_sh.at[pl.ds(c * CHK, CHK)])
        return carry
    lax.fori_loop(0, -(-(N // CHK) // NS), _zero_u, 0)
    pltpu.sync_copy(zero1d, deg_sh.at[pl.ds(t * 3200, 3200)])
    pltpu.sync_copy(cf_hbm, coeff_buf)
    plsc.subcore_barrier()

    def _load_sup_sync(s):
        off = base + s * SUP
        cps = []
        for rix in range(3):
            cps.append(pltpu.async_copy(
                e3_hbm.at[pl.ds(rix * E + off, SUP)],
                sup0.at[pl.ds(rix * SUP, SUP)], sem_ei0))
        for cp in cps:
            cp.wait()

    def _p1(u, carry):
        _load_sup_sync(u)
        descs = [None, None]
        for j in range(CPS):
            p = j & 1
            if descs[p] is not None:
                descs[p].wait()
            for seg in range(CHK // 16):
                cbase = j * CHK + seg * 16
                d = sup0[pl.ds(SUP + cbase, 16)]
                r = sup0[pl.ds(2 * SUP + cbase, 16)]
                idx[p][pl.ds(seg * 16, 16)] = r * N + d
            descs[p] = pltpu.async_copy(ones_buf, deg_sh.at[idx[p]],
                                        sem_d1[p], add=True)
        descs[0].wait()
        descs[1].wait()
        return carry
    lax.fori_loop(0, NSUP, _p1, 0)
    plsc.subcore_barrier()

    bvec = jnp.full((16,), b, jnp.int32)

    def _cidx(j, p):
        for seg in range(CHK // 16):
            cbase = j * CHK + seg * 16
            sv = sup0[pl.ds(cbase, 16)]
            d = sup0[pl.ds(SUP + cbase, 16)]
            r = sup0[pl.ds(2 * SUP + cbase, 16)]
            src[p][pl.ds(seg * 16, 16)] = sv
            dst[p][pl.ds(seg * 16, 16)] = d
            idx[p][pl.ds(seg * 16, 16)] = r * N + d
            cf = plsc.load_gather(coeff_buf, [r * NB + bvec])
            sb[p][pl.ds(seg * 16, 16)] = cf

    def _gath(p):
        return (pltpu.async_copy(deg_sh.at[idx[p]], dg[p], sem_dg[p]),
                pltpu.async_copy(x_hbm.at[src[p]], rows[p], sem_g[p]))

    def _p2(u, carry):
        _load_sup_sync(u)
        _cidx(0, 0)
        cps = [None, None]
        scs = [None, None]
        cps[0] = _gath(0)
        for j in range(CPS):
            p = j & 1
            pn = 1 - p
            if j + 1 < CPS:
                if scs[pn] is not None:
                    scs[pn].wait()
                _cidx(j + 1, pn)
                cps[pn] = _gath(pn)
            cps[p][0].wait()
            cps[p][1].wait()
            for seg in range(CHK // 16):
                sl = pl.ds(seg * 16, 16)
                sb[p][sl] = sb[p][sl] / jnp.maximum(dg[p][sl], 1.0)

            def _scale(e, p=p):
                svv = plsc.load_gather(
                    sb[p], [jnp.full((16,), e, jnp.int32)])
                for seg in range(8):
                    rows[p][e, pl.ds(seg * 16, 16)] = (
                        rows[p][e, pl.ds(seg * 16, 16)] * svv)
            plsc.parallel_loop(0, CHK, unroll=4)(_scale)
            scs[p] = pltpu.async_copy(rows[p], u_sh.at[dst[p]], sem_sc[p],
                                      add=True)
        scs[1].wait()
        scs[0].wait()
        return carry
    lax.fori_loop(0, NSUP, _p2, 0)
    plsc.subcore_barrier()

    def _ep(k, carry):
        c = k * NS + t

        @pl.when(c < N // CHK)
        def _():
            pltpu.sync_copy(u_sh.at[pl.ds(c * CHK, CHK)],
                            u_hbm.at[b, pl.ds(c * CHK, CHK)])
        return carry
    lax.fori_loop(0, -(-(N // CHK) // NS), _ep, 0)


_sc_call = pl.kernel(
    _sc_body,
    out_type=jax.ShapeDtypeStruct((NB, N, D), jnp.float32),
    mesh=plsc.VectorSubcoreMesh(core_axis_name="c", subcore_axis_name="s"),
    compiler_params=pltpu.CompilerParams(needs_layout_passes=False),
    scratch_types=[
        pltpu.VMEM((3 * SUP,), jnp.int32),
        pltpu.VMEM((CHK, D), jnp.float32),
        pltpu.VMEM((CHK, D), jnp.float32),
        pltpu.VMEM((CHK,), jnp.int32),
        pltpu.VMEM((CHK,), jnp.int32),
        pltpu.VMEM((CHK,), jnp.int32),
        pltpu.VMEM((CHK,), jnp.int32),
        pltpu.VMEM((CHK,), jnp.int32),
        pltpu.VMEM((CHK,), jnp.int32),
        pltpu.VMEM((CHK,), jnp.float32),
        pltpu.VMEM((CHK,), jnp.float32),
        pltpu.VMEM((CHK,), jnp.float32),
        pltpu.VMEM((CHK,), jnp.float32),
        pltpu.VMEM((CHK,), jnp.float32),
        pltpu.VMEM((16,), jnp.float32),
        pltpu.VMEM((3200,), jnp.float32),
        pltpu.VMEM_SHARED((N, D), jnp.float32),
        pltpu.VMEM_SHARED((DEG_PAD,), jnp.float32),
        pltpu.SemaphoreType.DMA,
        pltpu.SemaphoreType.DMA,
        pltpu.SemaphoreType.DMA,
        pltpu.SemaphoreType.DMA,
        pltpu.SemaphoreType.DMA,
        pltpu.SemaphoreType.DMA,
        pltpu.SemaphoreType.DMA,
        pltpu.SemaphoreType.DMA,
        pltpu.SemaphoreType.DMA,
    ],
)


BLK = 1000


def _tc_body(u_ref, x_ref, v_ref, wsl_ref, bias_ref, o_ref):
    acc = jnp.dot(u_ref[0], v_ref[0], preferred_element_type=jnp.float32)
    acc = acc + jnp.dot(u_ref[1], v_ref[1], preferred_element_type=jnp.float32)
    acc = acc + jnp.dot(x_ref[...], wsl_ref[...],
                        preferred_element_type=jnp.float32)
    o_ref[...] = acc + bias_ref[...]


_tc_call = pl.pallas_call(
    _tc_body,
    grid=(N // BLK,),
    in_specs=[
        pl.BlockSpec((NB, BLK, D), lambda i: (0, i, 0)),
        pl.BlockSpec((BLK, D), lambda i: (i, 0)),
        pl.BlockSpec((NB, D, D), lambda i: (0, 0, 0)),
        pl.BlockSpec((D, D), lambda i: (0, 0)),
        pl.BlockSpec((1, D), lambda i: (0, 0)),
    ],
    out_specs=pl.BlockSpec((BLK, D), lambda i: (i, 0)),
    out_shape=jax.ShapeDtypeStruct((N, D), jnp.float32),
)


@jax.jit
def kernel(x, edge_index, edge_type, basis_V, basis_coeff, bias,
           self_loop_weight):
    edges3 = jnp.concatenate([edge_index, edge_type[None, :]],
                             axis=0).reshape(-1)
    coeff_flat = jnp.pad(basis_coeff.reshape(-1), (0, 16 - R * NB))
    u = _sc_call(x, edges3, coeff_flat)
    return _tc_call(u, x, basis_V, self_loop_weight, bias.reshape(1, D))
